# baseline (speedup 1.0000x reference)
import jax
import jax.numpy as jnp
from jax import lax
from jax.experimental import pallas as pl
from jax.experimental.pallas import tpu as pltpu
from jax.experimental.pallas import tpu_sc as plsc

N = 10000
E = 320000
D = 128
W144 = 144
CH = 80
CHA = 48
CHB = 32
NW = 32
EPW = E // NW
NCH = EPW // CH
SGRP = 5
NG = NCH // SGRP
SB = SGRP * CH
RB = SGRP * (CH // 8)
STRIPE = 632
NPAD = 16 * STRIPE

_SC_PARAMS = pltpu.CompilerParams(
    needs_layout_passes=False, use_tc_tiling_on_sc=False)


def _mm_body(x_ref, w_ref, ka2_ref, h_ref, ab_ref):
    hb = jnp.dot(x_ref[...], w_ref[...], preferred_element_type=jnp.float32)
    h_ref[...] = hb
    ab_ref[...] = jnp.dot(hb, ka2_ref[...], preferred_element_type=jnp.float32)


def _score_body(srcf_hbm, dstf_hbm, asrc_hbm, adst_hbm,
                s_hbm, keep_hbm, rid_hbm, runn_hbm,
                srcf_v, dstf_v, asrc_v, adst_v, s_v, keep_v, rid_v, runn_v):
    c = lax.axis_index("c")
    s_id = lax.axis_index("s")
    base = (s_id * 2 + c) * EPW

    pltpu.sync_copy(srcf_hbm.at[pl.ds(base, EPW)], srcf_v)
    pltpu.sync_copy(dstf_hbm.at[pl.ds(base, EPW)], dstf_v)
    pltpu.sync_copy(asrc_hbm, asrc_v)
    pltpu.sync_copy(adst_hbm, adst_v)

    iota16 = lax.iota(jnp.int32, 16)
    junk16 = jnp.full((16,), NPAD - 1, jnp.int32)

    def _chunk(cc, _):
        c0 = cc * CH
        for v in range(CH // 16):
            runn_v[pl.ds(c0 + v * 16, 16)] = junk16

        def _vreg(v, carry):
            off = c0 + v * 16
            src16 = srcf_v[pl.ds(off, 16)]
            dst16 = dstf_v[pl.ds(off, 16)]
            raw = (plsc.load_gather(asrc_v, [src16])
                   + plsc.load_gather(adst_v, [dst16]))
            lk = jnp.maximum(raw, raw * 0.2)
            s_v[pl.ds(off, 16)] = jnp.exp(jnp.clip(lk, -2.0, 2.0))
            gidx = jnp.full((16,), off, jnp.int32) + iota16
            prev16 = plsc.load_gather(srcf_v, [jnp.maximum(gidx - 1, 0)])
            same = (src16 == prev16) & (gidx != c0)
            keep_v[pl.ds(off, 16)] = jnp.where(same, 1.0, 0.0)
            b16 = jnp.where(same, 0, 1)
            rid16 = plsc.cumsum(b16) + carry
            rid_v[pl.ds(off, 16)] = rid16
            plsc.store_scatter(
                runn_v, [jnp.full((16,), c0, jnp.int32) + rid16], src16,
                mask=~same)
            return rid16[15]

        lax.fori_loop(0, CH // 16, _vreg, jnp.int32(-1))
        return 0

    lax.fori_loop(0, NCH, _chunk, 0)

    pltpu.sync_copy(s_v, s_hbm.at[pl.ds(base, EPW)])
    pltpu.sync_copy(keep_v, keep_hbm.at[pl.ds(base, EPW)])
    pltpu.sync_copy(rid_v, rid_hbm.at[pl.ds(base, EPW)])
    pltpu.sync_copy(runn_v, runn_hbm.at[pl.ds(base, EPW)])


def _agg_body(h_hbm, dstf_hbm, s_hbm, keep_hbm, rid_hbm, runn2_hbm,
              acc0_hbm, acc1_hbm,
              dstg_v, sg_v, keepg_v, ridg_v, runng_v, rowsa_v, rowsb_v,
              flush_v, acc_sh, sem_a, sem_b, sem_pf):
    c = lax.axis_index("c")
    s_id = lax.axis_index("s")
    w = s_id * 2 + c
    base = w * EPW
    rbase = w * (EPW // 8)

    iota16 = lax.iota(jnp.int32, 16)
    e0 = jnp.where(iota16 == 0, 1.0, 0.0)

    def _zrow(k, _):
        for i in range(W144 // 16):
            flush_v[k, pl.ds(i * 16, 16)] = jnp.zeros((16,), jnp.float32)
        return 0
    lax.fori_loop(0, CH, _zrow, 0)
    row0 = s_id * STRIPE
    for off in range(0, STRIPE - 72, 80):
        pltpu.sync_copy(flush_v.at[pl.ds(0, 80)],
                        acc_sh.at[pl.ds(row0 + off, 80)])
    pltpu.sync_copy(flush_v.at[pl.ds(0, 72)],
                    acc_sh.at[pl.ds(row0 + STRIPE - 72, 72)])

    plsc.subcore_barrier()

    def _stage(g, par_off, rpar_off, sync):
        cpys = [
            (dstf_hbm.at[pl.ds(base + g * SB, SB)],
             dstg_v.at[pl.ds(par_off, SB)]),
            (s_hbm.at[pl.ds(base + g * SB, SB)],
             sg_v.at[pl.ds(par_off, SB)]),
            (keep_hbm.at[pl.ds(base + g * SB, SB)],
             keepg_v.at[pl.ds(par_off, SB)]),
            (rid_hbm.at[pl.ds(base + g * SB, SB)],
             ridg_v.at[pl.ds(par_off, SB)]),
            (runn2_hbm.at[pl.ds(rbase + g * RB, RB)],
             runng_v.at[pl.ds(rpar_off, RB)]),
        ]
        for s_ref, d_ref in cpys:
            if sync:
                pltpu.sync_copy(s_ref, d_ref)
            else:
                pltpu.async_copy(s_ref, d_ref, sem_pf)

    def _drain_stage():
        for buf, n in ((dstg_v, SB), (sg_v, SB), (keepg_v, SB), (ridg_v, SB)):
            pltpu.make_async_copy(s_hbm.at[pl.ds(0, n)],
                                  buf.at[pl.ds(0, n)], sem_pf).wait()
        pltpu.make_async_copy(runn2_hbm.at[pl.ds(0, RB)],
                              runng_v.at[pl.ds(0, RB)], sem_pf).wait()

    def _wait_rows(rows, sem, nrows):
        pltpu.make_async_copy(h_hbm.at[pl.ds(0, nrows)], rows, sem).wait()

    def _edge_loop(nrows, rows, soff, acc):
        def body(k, acc):
            off16 = jnp.full((16,), soff, jnp.int32) + k
            sc = plsc.load_gather(sg_v, [off16])
            kp = plsc.load_gather(keepg_v, [off16])
            r0 = plsc.load_gather(ridg_v, [off16])[0]
            new = []
            for i in range(8):
                sl = pl.ds(i * 16, 16)
                new.append(acc[i] * kp + sc * rows[k, sl])
            new.append(acc[8] * kp + sc * e0)
            for i in range(W144 // 16):
                flush_v[r0, pl.ds(i * 16, 16)] = new[i]
            return tuple(new)
        return lax.fori_loop(0, nrows, body, acc)

    zacc = tuple(jnp.zeros((16,), jnp.float32) for _ in range(W144 // 16))

    _stage(0, 0, 0, True)
    pltpu.async_copy(h_hbm.at[dstg_v.at[pl.ds(0, CHA)]], rowsa_v, sem_a)

    def _group(g, _):
        par = pl.multiple_of((g % 2) * SB, 8)
        rpar = (g % 2) * RB

        @pl.when(g < NG - 1)
        def _():
            _stage(g + 1, pl.multiple_of(((g + 1) % 2) * SB, 8),
                   ((g + 1) % 2) * RB, False)

        for jj in range(SGRP):
            soff = par + jj * CH
            _wait_rows(rowsa_v, sem_a, CHA)
            pltpu.async_copy(
                h_hbm.at[dstg_v.at[pl.ds(soff + CHA, CHB)]], rowsb_v, sem_b)
            acc = _edge_loop(CHA, rowsa_v, soff, zacc)
            _wait_rows(rowsb_v, sem_b, CHB)
            if jj < SGRP - 1:
                pltpu.async_copy(
                    h_hbm.at[dstg_v.at[pl.ds(soff + CH, CHA)]],
                    rowsa_v, sem_a)
            acc = _edge_loop(CHB, rowsb_v, soff + CHA, acc)
            nruns = plsc.load_gather(
                ridg_v, [jnp.full((16,), soff + CH - 1, jnp.int32)])[0] + 1
            nblk = (nruns + 7) // 8

            for b in range(CH // 8):
                @pl.when(b < nblk)
                def _():
                    pltpu.sync_copy(
                        flush_v.at[pl.ds(b * 8, 8)],
                        acc_sh.at[runng_v.at[rpar + jj * (CH // 8) + b]],
                        add=True)

        @pl.when(g < NG - 1)
        def _():
            _drain_stage()
            nxt = pl.multiple_of(((g + 1) % 2) * SB, 8)
            pltpu.async_copy(
                h_hbm.at[dstg_v.at[pl.ds(nxt, CHA)]], rowsa_v, sem_a)
        return 0

    lax.fori_loop(0, NG, _group, 0)

    plsc.subcore_barrier()

    @pl.when(c == 0)
    def _():
        pltpu.sync_copy(acc_sh.at[pl.ds(row0, STRIPE)],
                        acc0_hbm.at[pl.ds(row0, STRIPE)])

    @pl.when(c == 1)
    def _():
        pltpu.sync_copy(acc_sh.at[pl.ds(row0, STRIPE)],
                        acc1_hbm.at[pl.ds(row0, STRIPE)])


def _combine_body(a0_ref, a1_ref, out_ref):
    t = a0_ref[...] + a1_ref[...]
    num = t[:, 0:D]
    den = t[:, D:D + 1]
    safe = jnp.where(den > 0.0, den, 1.0)
    out_ref[...] = num / safe


def kernel(node_states, edges, kernel, kernel_attention):
    ka2 = jnp.concatenate(
        [kernel_attention[:D], kernel_attention[D:]], axis=1)

    blk = 2000
    h, ab = pl.pallas_call(
        _mm_body,
        grid=(N // blk,),
        in_specs=[
            pl.BlockSpec((blk, D), lambda i: (i, 0)),
            pl.BlockSpec((D, D), lambda i: (0, 0)),
            pl.BlockSpec((D, 2), lambda i: (0, 0)),
        ],
        out_specs=[
            pl.BlockSpec((blk, D), lambda i: (i, 0)),
            pl.BlockSpec((blk, 2), lambda i: (i, 0)),
        ],
        out_shape=[
            jax.ShapeDtypeStruct((N, D), jnp.float32),
            jax.ShapeDtypeStruct((N, 2), jnp.float32),
        ],
    )(node_states, kernel, ka2)

    src = edges[:, 0]
    dst = edges[:, 1]

    mesh = plsc.VectorSubcoreMesh(core_axis_name="c", subcore_axis_name="s")

    s_all, keep_all, rid_all, runn_all = pl.kernel(
        _score_body,
        out_type=[
            jax.ShapeDtypeStruct((E,), jnp.float32),
            jax.ShapeDtypeStruct((E,), jnp.float32),
            jax.ShapeDtypeStruct((E,), jnp.int32),
            jax.ShapeDtypeStruct((E,), jnp.int32),
        ],
        mesh=mesh,
        compiler_params=_SC_PARAMS,
        scratch_types=[
            pltpu.VMEM((EPW,), jnp.int32),
            pltpu.VMEM((EPW,), jnp.int32),
            pltpu.VMEM((N,), jnp.float32),
            pltpu.VMEM((N,), jnp.float32),
            pltpu.VMEM((EPW,), jnp.float32),
            pltpu.VMEM((EPW,), jnp.float32),
            pltpu.VMEM((EPW,), jnp.int32),
            pltpu.VMEM((EPW,), jnp.int32),
        ],
    )(src, dst, ab[:, 0], ab[:, 1])

    runn2 = runn_all.reshape(E // 8, 8)

    acc0, acc1 = pl.kernel(
        _agg_body,
        out_type=[
            jax.ShapeDtypeStruct((NPAD, W144), jnp.float32),
            jax.ShapeDtypeStruct((NPAD, W144), jnp.float32),
        ],
        mesh=mesh,
        compiler_params=_SC_PARAMS,
        scratch_types=[
            pltpu.VMEM((2 * SB,), jnp.int32),
            pltpu.VMEM((2 * SB,), jnp.float32),
            pltpu.VMEM((2 * SB,), jnp.float32),
            pltpu.VMEM((2 * SB,), jnp.int32),
            pltpu.VMEM((2 * RB, 8), jnp.int32),
            pltpu.VMEM((CHA, D), jnp.float32),
            pltpu.VMEM((CHB, D), jnp.float32),
            pltpu.VMEM((CH, W144), jnp.float32),
            pltpu.VMEM_SHARED((NPAD, W144), jnp.float32),
            pltpu.SemaphoreType.DMA,
            pltpu.SemaphoreType.DMA,
            pltpu.SemaphoreType.DMA,
        ],
    )(h, dst, s_all, keep_all, rid_all, runn2)

    out = pl.pallas_call(
        _combine_body,
        grid=(N // blk,),
        in_specs=[
            pl.BlockSpec((blk, W144), lambda i: (i, 0)),
            pl.BlockSpec((blk, W144), lambda i: (i, 0)),
        ],
        out_specs=pl.BlockSpec((blk, D), lambda i: (i, 0)),
        out_shape=jax.ShapeDtypeStruct((N, D), jnp.float32),
    )(acc0, acc1)
    return out


# speedup vs baseline: 1.1593x; 1.1593x over previous
import jax
import jax.numpy as jnp
from jax import lax
from jax.experimental import pallas as pl
from jax.experimental.pallas import tpu as pltpu
from jax.experimental.pallas import tpu_sc as plsc

N = 10000
E = 320000
D = 128
W144 = 144
CH = 50
NW = 32
EPW = E // NW
NCH = EPW // CH
SGRP = 8
NG = NCH // SGRP
STRIPE = 632
NPAD = 16 * STRIPE

_SC_PARAMS = pltpu.CompilerParams(
    needs_layout_passes=False, use_tc_tiling_on_sc=False)


def _ab_body(x_ref, w_ref, ka2_ref, ab_ref):
    wk = jnp.dot(w_ref[...], ka2_ref[...], preferred_element_type=jnp.float32)
    ab_ref[...] = jnp.dot(x_ref[...], wk, preferred_element_type=jnp.float32)


def _mm_body(x_ref, w_ref, h_ref):
    hb = jnp.dot(x_ref[...], w_ref[...], preferred_element_type=jnp.float32)
    h_ref[:, 0:D] = hb
    col = lax.broadcasted_iota(jnp.int32, (hb.shape[0], 16), 1)
    h_ref[:, D:W144] = jnp.where(col == 0, 1.0, 0.0)


def _score_body(srcf_hbm, dstf_hbm, asrc_hbm, adst_hbm, s_hbm,
                srcf_v, dstf_v, asrc_v, adst_v, s_v):
    c = lax.axis_index("c")
    s_id = lax.axis_index("s")
    base = (s_id * 2 + c) * EPW

    pltpu.sync_copy(srcf_hbm.at[pl.ds(base, EPW)], srcf_v)
    pltpu.sync_copy(dstf_hbm.at[pl.ds(base, EPW)], dstf_v)
    pltpu.sync_copy(asrc_hbm, asrc_v)
    pltpu.sync_copy(adst_hbm, adst_v)

    def _score(i, _):
        si = srcf_v[pl.ds(i * 16, 16)]
        di = dstf_v[pl.ds(i * 16, 16)]
        raw = plsc.load_gather(asrc_v, [si]) + plsc.load_gather(adst_v, [di])
        lk = jnp.maximum(raw, raw * 0.2)
        s_v[pl.ds(i * 16, 16)] = jnp.exp(jnp.clip(lk, -2.0, 2.0))
        return 0
    lax.fori_loop(0, EPW // 16, _score, 0)

    pltpu.sync_copy(s_v, s_hbm.at[pl.ds(base, EPW)])


def _agg_body(h_hbm, src2_hbm, dst2_hbm, s_hbm, acc0_hbm, acc1_hbm,
              src2_v, dst2_v, sg_v, rows0_v, rows1_v, acc_sh,
              sem_g0, sem_g1, sem_s0, sem_s1, sem_sg):
    c = lax.axis_index("c")
    s_id = lax.axis_index("s")
    w = s_id * 2 + c
    base = w * EPW
    rows = (rows0_v, rows1_v)
    sem_g = (sem_g0, sem_g1)
    sem_s = (sem_s0, sem_s1)
    SB = SGRP * CH

    pltpu.sync_copy(src2_hbm.at[pl.ds(w * NCH, NCH)], src2_v)
    pltpu.sync_copy(dst2_hbm.at[pl.ds(w * NCH, NCH)], dst2_v)

    def _zrow(k, _):
        for i in range(W144 // 16):
            rows0_v[k, pl.ds(i * 16, 16)] = jnp.zeros((16,), jnp.float32)
        return 0
    lax.fori_loop(0, CH, _zrow, 0)
    row0 = s_id * STRIPE
    for off in range(0, STRIPE - 8, 48):
        pltpu.sync_copy(rows0_v.at[pl.ds(0, 48)],
                        acc_sh.at[pl.ds(row0 + off, 48)])
    pltpu.sync_copy(rows0_v.at[pl.ds(0, 8)],
                    acc_sh.at[pl.ds(row0 + STRIPE - 8, 8)])

    plsc.subcore_barrier()

    def _wait_gather(p):
        pltpu.make_async_copy(h_hbm.at[pl.ds(0, CH)], rows[p],
                              sem_g[p]).wait()

    def _wait_scatter(p):
        pltpu.make_async_copy(rows[p], acc_sh.at[pl.ds(0, CH)],
                              sem_s[p]).wait()

    pltpu.sync_copy(s_hbm.at[pl.ds(base, SB)], sg_v.at[pl.ds(0, SB)])
    pltpu.async_copy(h_hbm.at[dst2_v.at[0]], rows0_v, sem_g0)

    def _group(g, _):
        nxt_off = pl.multiple_of(((g + 1) % 2) * SB, 8)

        @pl.when(g < NG - 1)
        def _():
            pltpu.async_copy(
                s_hbm.at[pl.ds(base + (g + 1) * SB, SB)],
                sg_v.at[pl.ds(nxt_off, SB)], sem_sg)

        s_off = (g % 2) * SB
        for jj in range(SGRP):
            j = g * SGRP + jj
            p = jj % 2
            q = 1 - p
            _wait_gather(p)
            if jj == 0:
                @pl.when(g > 0)
                def _():
                    _wait_scatter(q)
                pltpu.async_copy(h_hbm.at[dst2_v.at[j + 1]], rows[q],
                                 sem_g[q])
            elif jj < SGRP - 1:
                _wait_scatter(q)
                pltpu.async_copy(h_hbm.at[dst2_v.at[j + 1]], rows[q],
                                 sem_g[q])
            else:
                @pl.when(g < NG - 1)
                def _():
                    _wait_scatter(q)
                    pltpu.async_copy(h_hbm.at[dst2_v.at[j + 1]], rows[q],
                                     sem_g[q])

            base16 = jnp.full((16,), s_off + jj * CH, jnp.int32)

            def _scale(k2, _):
                k = k2 * 2
                sc0 = plsc.load_gather(sg_v, [base16 + k])
                sc1 = plsc.load_gather(sg_v, [base16 + (k + 1)])
                for i in range(W144 // 16):
                    sl = pl.ds(i * 16, 16)
                    rows[p][k, sl] = rows[p][k, sl] * sc0
                    rows[p][k + 1, sl] = rows[p][k + 1, sl] * sc1
                return 0
            lax.fori_loop(0, CH // 2, _scale, 0)
            pltpu.async_copy(rows[p], acc_sh.at[src2_v.at[j]], sem_s[p],
                             add=True)

        @pl.when(g < NG - 1)
        def _():
            pltpu.make_async_copy(s_hbm.at[pl.ds(0, SB)],
                                  sg_v.at[pl.ds(0, SB)], sem_sg).wait()
        return 0

    lax.fori_loop(0, NG, _group, 0)

    _wait_scatter(0)
    _wait_scatter(1)

    plsc.subcore_barrier()

    @pl.when(c == 0)
    def _():
        pltpu.sync_copy(acc_sh.at[pl.ds(row0, STRIPE)],
                        acc0_hbm.at[pl.ds(row0, STRIPE)])

    @pl.when(c == 1)
    def _():
        pltpu.sync_copy(acc_sh.at[pl.ds(row0, STRIPE)],
                        acc1_hbm.at[pl.ds(row0, STRIPE)])


def _combine_body(a0_ref, a1_ref, out_ref):
    t = a0_ref[...] + a1_ref[...]
    num = t[:, 0:D]
    den = t[:, D:D + 1]
    safe = jnp.where(den > 0.0, den, 1.0)
    out_ref[...] = num / safe


def kernel(node_states, edges, kernel, kernel_attention):
    ka2 = jnp.concatenate(
        [kernel_attention[:D], kernel_attention[D:]], axis=1)

    blk = 2000
    ab = pl.pallas_call(
        _ab_body,
        grid=(N // blk,),
        in_specs=[
            pl.BlockSpec((blk, D), lambda i: (i, 0)),
            pl.BlockSpec((D, D), lambda i: (0, 0)),
            pl.BlockSpec((D, 2), lambda i: (0, 0)),
        ],
        out_specs=pl.BlockSpec((blk, 2), lambda i: (i, 0)),
        out_shape=jax.ShapeDtypeStruct((N, 2), jnp.float32),
    )(node_states, kernel, ka2)
    h144 = pl.pallas_call(
        _mm_body,
        grid=(N // blk,),
        in_specs=[
            pl.BlockSpec((blk, D), lambda i: (i, 0)),
            pl.BlockSpec((D, D), lambda i: (0, 0)),
        ],
        out_specs=pl.BlockSpec((blk, W144), lambda i: (i, 0)),
        out_shape=jax.ShapeDtypeStruct((N, W144), jnp.float32),
    )(node_states, kernel)

    src = edges[:, 0]
    dst = edges[:, 1]
    src2 = src.reshape(E // CH, CH)
    dst2 = dst.reshape(E // CH, CH)

    mesh = plsc.VectorSubcoreMesh(core_axis_name="c", subcore_axis_name="s")

    s_all = pl.kernel(
        _score_body,
        out_type=jax.ShapeDtypeStruct((E,), jnp.float32),
        mesh=mesh,
        compiler_params=_SC_PARAMS,
        scratch_types=[
            pltpu.VMEM((EPW,), jnp.int32),
            pltpu.VMEM((EPW,), jnp.int32),
            pltpu.VMEM((N,), jnp.float32),
            pltpu.VMEM((N,), jnp.float32),
            pltpu.VMEM((EPW,), jnp.float32),
        ],
    )(src, dst, ab[:, 0], ab[:, 1])

    acc0, acc1 = pl.kernel(
        _agg_body,
        out_type=[
            jax.ShapeDtypeStruct((NPAD, W144), jnp.float32),
            jax.ShapeDtypeStruct((NPAD, W144), jnp.float32),
        ],
        mesh=mesh,
        compiler_params=_SC_PARAMS,
        scratch_types=[
            pltpu.VMEM((NCH, CH), jnp.int32),
            pltpu.VMEM((NCH, CH), jnp.int32),
            pltpu.VMEM((2 * SGRP * CH,), jnp.float32),
            pltpu.VMEM((CH, W144), jnp.float32),
            pltpu.VMEM((CH, W144), jnp.float32),
            pltpu.VMEM_SHARED((NPAD, W144), jnp.float32),
            pltpu.SemaphoreType.DMA,
            pltpu.SemaphoreType.DMA,
            pltpu.SemaphoreType.DMA,
            pltpu.SemaphoreType.DMA,
            pltpu.SemaphoreType.DMA,
        ],
    )(h144, src2, dst2, s_all)

    out = pl.pallas_call(
        _combine_body,
        grid=(N // blk,),
        in_specs=[
            pl.BlockSpec((blk, W144), lambda i: (i, 0)),
            pl.BlockSpec((blk, W144), lambda i: (i, 0)),
        ],
        out_specs=pl.BlockSpec((blk, D), lambda i: (i, 0)),
        out_shape=jax.ShapeDtypeStruct((N, D), jnp.float32),
    )(acc0, acc1)
    return out


# speedup vs baseline: 1.1900x; 1.0265x over previous
import jax
import jax.numpy as jnp
from jax import lax
from jax.experimental import pallas as pl
from jax.experimental.pallas import tpu as pltpu
from jax.experimental.pallas import tpu_sc as plsc

N = 10000
E = 320000
D = 128
W144 = 144
CH = 50
NW = 32
EPW = E // NW
NCH = EPW // CH
SGRP = 8
NG = NCH // SGRP
STRIPE = 632
NPAD = 16 * STRIPE

_SC_PARAMS = pltpu.CompilerParams(
    needs_layout_passes=False, use_tc_tiling_on_sc=False)


def _mm_body(x_ref, w_ref, ka2_ref, h_ref, ab_ref):
    hb = jnp.dot(x_ref[...], w_ref[...], preferred_element_type=jnp.float32)
    h_ref[:, 0:D] = hb
    col = lax.broadcasted_iota(jnp.int32, (hb.shape[0], 16), 1)
    h_ref[:, D:W144] = jnp.where(col == 0, 1.0, 0.0)
    ab_ref[...] = jnp.dot(hb, ka2_ref[...], preferred_element_type=jnp.float32)


def _score_body(srcf_hbm, dstf_hbm, asrc_hbm, adst_hbm, s_hbm,
                srcf_v, dstf_v, asrc_v, adst_v, s_v):
    c = lax.axis_index("c")
    s_id = lax.axis_index("s")
    base = (s_id * 2 + c) * EPW

    pltpu.sync_copy(srcf_hbm.at[pl.ds(base, EPW)], srcf_v)
    pltpu.sync_copy(dstf_hbm.at[pl.ds(base, EPW)], dstf_v)
    pltpu.sync_copy(asrc_hbm, asrc_v)
    pltpu.sync_copy(adst_hbm, adst_v)

    def _score(i, _):
        si = srcf_v[pl.ds(i * 16, 16)]
        di = dstf_v[pl.ds(i * 16, 16)]
        raw = plsc.load_gather(asrc_v, [si]) + plsc.load_gather(adst_v, [di])
        lk = jnp.maximum(raw, raw * 0.2)
        s_v[pl.ds(i * 16, 16)] = jnp.exp(jnp.clip(lk, -2.0, 2.0))
        return 0
    lax.fori_loop(0, EPW // 16, _score, 0)

    pltpu.sync_copy(s_v, s_hbm.at[pl.ds(base, EPW)])


def _agg_body(h_hbm, src2_hbm, dst2_hbm, s_hbm, acc0_hbm, acc1_hbm,
              src2_v, dst2_v, sg_v, rows0_v, rows1_v, acc_sh,
              sem_g0, sem_g1, sem_s0, sem_s1, sem_sg):
    c = lax.axis_index("c")
    s_id = lax.axis_index("s")
    w = s_id * 2 + c
    base = w * EPW
    rows = (rows0_v, rows1_v)
    sem_g = (sem_g0, sem_g1)
    sem_s = (sem_s0, sem_s1)
    SB = SGRP * CH

    pltpu.sync_copy(src2_hbm.at[pl.ds(w * NCH, NCH)], src2_v)
    pltpu.sync_copy(dst2_hbm.at[pl.ds(w * NCH, NCH)], dst2_v)

    def _zrow(k, _):
        for i in range(W144 // 16):
            rows0_v[k, pl.ds(i * 16, 16)] = jnp.zeros((16,), jnp.float32)
        return 0
    lax.fori_loop(0, CH, _zrow, 0)
    row0 = s_id * STRIPE
    for off in range(0, STRIPE - 8, 48):
        pltpu.sync_copy(rows0_v.at[pl.ds(0, 48)],
                        acc_sh.at[pl.ds(row0 + off, 48)])
    pltpu.sync_copy(rows0_v.at[pl.ds(0, 8)],
                    acc_sh.at[pl.ds(row0 + STRIPE - 8, 8)])

    plsc.subcore_barrier()

    def _wait_gather(p):
        pltpu.make_async_copy(h_hbm.at[pl.ds(0, CH)], rows[p],
                              sem_g[p]).wait()

    def _wait_scatter(p):
        pltpu.make_async_copy(rows[p], acc_sh.at[pl.ds(0, CH)],
                              sem_s[p]).wait()

    pltpu.sync_copy(s_hbm.at[pl.ds(base, SB)], sg_v.at[pl.ds(0, SB)])
    pltpu.async_copy(h_hbm.at[dst2_v.at[0]], rows0_v, sem_g0)

    def _group(g, _):
        nxt_off = pl.multiple_of(((g + 1) % 2) * SB, 8)

        @pl.when(g < NG - 1)
        def _():
            pltpu.async_copy(
                s_hbm.at[pl.ds(base + (g + 1) * SB, SB)],
                sg_v.at[pl.ds(nxt_off, SB)], sem_sg)

        s_off = (g % 2) * SB
        for jj in range(SGRP):
            j = g * SGRP + jj
            p = jj % 2
            q = 1 - p
            _wait_gather(p)
            if jj == 0:
                @pl.when(g > 0)
                def _():
                    _wait_scatter(q)
                pltpu.async_copy(h_hbm.at[dst2_v.at[j + 1]], rows[q],
                                 sem_g[q])
            elif jj < SGRP - 1:
                _wait_scatter(q)
                pltpu.async_copy(h_hbm.at[dst2_v.at[j + 1]], rows[q],
                                 sem_g[q])
            else:
                @pl.when(g < NG - 1)
                def _():
                    _wait_scatter(q)
                    pltpu.async_copy(h_hbm.at[dst2_v.at[j + 1]], rows[q],
                                     sem_g[q])

            base16 = jnp.full((16,), s_off + jj * CH, jnp.int32)

            def _scale(k2, _):
                k = k2 * 2
                sc0 = plsc.load_gather(sg_v, [base16 + k])
                sc1 = plsc.load_gather(sg_v, [base16 + (k + 1)])
                for i in range(W144 // 16):
                    sl = pl.ds(i * 16, 16)
                    rows[p][k, sl] = rows[p][k, sl] * sc0
                    rows[p][k + 1, sl] = rows[p][k + 1, sl] * sc1
                return 0
            lax.fori_loop(0, CH // 2, _scale, 0)
            pltpu.async_copy(rows[p], acc_sh.at[src2_v.at[j]], sem_s[p],
                             add=True)

        @pl.when(g < NG - 1)
        def _():
            pltpu.make_async_copy(s_hbm.at[pl.ds(0, SB)],
                                  sg_v.at[pl.ds(0, SB)], sem_sg).wait()
        return 0

    lax.fori_loop(0, NG, _group, 0)

    _wait_scatter(0)
    _wait_scatter(1)

    plsc.subcore_barrier()

    @pl.when(c == 0)
    def _():
        pltpu.sync_copy(acc_sh.at[pl.ds(row0, STRIPE)],
                        acc0_hbm.at[pl.ds(row0, STRIPE)])

    @pl.when(c == 1)
    def _():
        pltpu.sync_copy(acc_sh.at[pl.ds(row0, STRIPE)],
                        acc1_hbm.at[pl.ds(row0, STRIPE)])


def _combine_body(a0_ref, a1_ref, out_ref):
    t = a0_ref[...] + a1_ref[...]
    num = t[:, 0:D]
    den = t[:, D:D + 1]
    safe = jnp.where(den > 0.0, den, 1.0)
    out_ref[...] = num / safe


def kernel(node_states, edges, kernel, kernel_attention):
    ka2 = jnp.concatenate(
        [kernel_attention[:D], kernel_attention[D:]], axis=1)

    blk = 2000
    h144, ab = pl.pallas_call(
        _mm_body,
        grid=(N // blk,),
        in_specs=[
            pl.BlockSpec((blk, D), lambda i: (i, 0)),
            pl.BlockSpec((D, D), lambda i: (0, 0)),
            pl.BlockSpec((D, 2), lambda i: (0, 0)),
        ],
        out_specs=[
            pl.BlockSpec((blk, W144), lambda i: (i, 0)),
            pl.BlockSpec((blk, 2), lambda i: (i, 0)),
        ],
        out_shape=[
            jax.ShapeDtypeStruct((N, W144), jnp.float32),
            jax.ShapeDtypeStruct((N, 2), jnp.float32),
        ],
    )(node_states, kernel, ka2)

    src = edges[:, 0]
    dst = edges[:, 1]
    src2 = src.reshape(E // CH, CH)
    dst2 = dst.reshape(E // CH, CH)

    mesh = plsc.VectorSubcoreMesh(core_axis_name="c", subcore_axis_name="s")

    s_all = pl.kernel(
        _score_body,
        out_type=jax.ShapeDtypeStruct((E,), jnp.float32),
        mesh=mesh,
        compiler_params=_SC_PARAMS,
        scratch_types=[
            pltpu.VMEM((EPW,), jnp.int32),
            pltpu.VMEM((EPW,), jnp.int32),
            pltpu.VMEM((N,), jnp.float32),
            pltpu.VMEM((N,), jnp.float32),
            pltpu.VMEM((EPW,), jnp.float32),
        ],
    )(src, dst, ab[:, 0], ab[:, 1])

    acc0, acc1 = pl.kernel(
        _agg_body,
        out_type=[
            jax.ShapeDtypeStruct((NPAD, W144), jnp.float32),
            jax.ShapeDtypeStruct((NPAD, W144), jnp.float32),
        ],
        mesh=mesh,
        compiler_params=_SC_PARAMS,
        scratch_types=[
            pltpu.VMEM((NCH, CH), jnp.int32),
            pltpu.VMEM((NCH, CH), jnp.int32),
            pltpu.VMEM((2 * SGRP * CH,), jnp.float32),
            pltpu.VMEM((CH, W144), jnp.float32),
            pltpu.VMEM((CH, W144), jnp.float32),
            pltpu.VMEM_SHARED((NPAD, W144), jnp.float32),
            pltpu.SemaphoreType.DMA,
            pltpu.SemaphoreType.DMA,
            pltpu.SemaphoreType.DMA,
            pltpu.SemaphoreType.DMA,
            pltpu.SemaphoreType.DMA,
        ],
    )(h144, src2, dst2, s_all)

    out = pl.pallas_call(
        _combine_body,
        grid=(N // blk,),
        in_specs=[
            pl.BlockSpec((blk, W144), lambda i: (i, 0)),
            pl.BlockSpec((blk, W144), lambda i: (i, 0)),
        ],
        out_specs=pl.BlockSpec((blk, D), lambda i: (i, 0)),
        out_shape=jax.ShapeDtypeStruct((N, D), jnp.float32),
    )(acc0, acc1)
    return out
